# baseline (device time: 15324 ns/iter reference)
import jax
import jax.numpy as jnp
from jax import lax
from jax.experimental import pallas as pl
from jax.experimental.pallas import tpu as pltpu

P = 64
BS = 16
NK = P * BS
B = 8
H = 8
D = 64
BH = B * H
NB = 64
SCALE = D ** -0.5
NEG = -1e30


def kernel(Q, K, V, bt, lens):
    K3 = K.reshape(NK * H, D)
    V3 = V.reshape(NK * H, D)
    Qf = Q.reshape(BH, D)
    lens2 = lens.reshape(B, 1)

    def body(k_ref, v_ref, q_ref, bt_ref, lens_ref, out_ref,
             psend, precv, send_sem, recv_sem):
        mx = lax.axis_index("x")
        my = lax.axis_index("y")
        mz = lax.axis_index("z")
        x_nbr = (1 - mx, my, mz)

        barrier_sem = pltpu.get_barrier_semaphore()
        pl.semaphore_signal(barrier_sem, inc=1, device_id=x_nbr,
                            device_id_type=pl.DeviceIdType.MESH)

        e8 = (
            lax.broadcasted_iota(jnp.int32, (B, BH), 1) // H
            == lax.broadcasted_iota(jnp.int32, (B, BH), 0)
        ).astype(jnp.float32)
        rep = (
            lax.broadcasted_iota(jnp.int32, (NK, P), 0) // BS
            == lax.broadcasted_iota(jnp.int32, (NK, P), 1)
        ).astype(jnp.float32)
        i64 = (
            lax.broadcasted_iota(jnp.int32, (BH, BH), 0)
            == lax.broadcasted_iota(jnp.int32, (BH, BH), 1)
        ).astype(jnp.float32)
        mask3 = (
            lax.broadcasted_iota(jnp.int32, (NK, H, BH), 1)
            == lax.broadcasted_iota(jnp.int32, (NK, H, BH), 2) % H
        ).astype(jnp.float32)

        pg = lax.broadcasted_iota(jnp.int32, (P, B, NB), 0) + mx * P
        slot = lax.broadcasted_iota(jnp.int32, (P, B, NB), 2)
        btv = bt_ref[...][None]
        lv = lens_ref[...][None]
        match = (btv == pg) & (slot < lv)
        c_pages = jnp.sum(match.astype(jnp.float32), axis=2)
        c_ab = _dot(rep, c_pages)
        c_cols = _dot(c_ab, e8)

        s3 = jax.lax.dot_general(
            k_ref[...], q_ref[...], (((1,), (1,)), ((), ())),
            preferred_element_type=jnp.float32,
        )
        s_bh = jnp.sum(s3.reshape(NK, H, BH) * mask3, axis=1) * SCALE
        s_m = jnp.where(c_cols > 0.0, s_bh, NEG)

        m_cols = jnp.max(s_m, axis=0, keepdims=True)
        e = jnp.exp(s_m - m_cols)
        w = c_cols * e
        l_cols = jnp.sum(w, axis=0, keepdims=True)

        w3 = (jnp.broadcast_to(w[:, None, :], (NK, H, BH)) * mask3).reshape(
            NK * H, BH
        )
        acc = jax.lax.dot_general(
            w3, v_ref[...], (((0,), (0,)), ((), ())),
            preferred_element_type=jnp.float32,
        )

        mt = jax.lax.dot_general(
            i64, m_cols, (((1,), (1,)), ((), ())),
            preferred_element_type=jnp.float32,
        )
        lt = jax.lax.dot_general(
            i64, l_cols, (((1,), (1,)), ((), ())),
            preferred_element_type=jnp.float32,
        )

        psend[0] = acc
        psend[1] = jnp.broadcast_to(mt, (BH, D))
        psend[2] = jnp.broadcast_to(lt, (BH, D))

        pl.semaphore_wait(barrier_sem, 1)
        rdma = pltpu.make_async_remote_copy(
            src_ref=psend, dst_ref=precv,
            send_sem=send_sem, recv_sem=recv_sem,
            device_id=x_nbr, device_id_type=pl.DeviceIdType.MESH,
        )
        rdma.start()
        rdma.wait()

        acc0, m0, l0 = psend[0], psend[1], psend[2]
        acc1, m1, l1 = precv[0], precv[1], precv[2]
        m_new = jnp.maximum(m0, m1)
        e0 = jnp.exp(m0 - m_new)
        e1 = jnp.exp(m1 - m_new)
        out_ref[...] = (acc0 * e0 + acc1 * e1) / (l0 * e0 + l1 * e1)

    out_shape = jax.ShapeDtypeStruct((BH, D), jnp.float32)
    res = pl.pallas_call(
        body,
        out_shape=out_shape,
        in_specs=[
            pl.BlockSpec(memory_space=pltpu.VMEM),
            pl.BlockSpec(memory_space=pltpu.VMEM),
            pl.BlockSpec(memory_space=pltpu.VMEM),
            pl.BlockSpec(memory_space=pltpu.VMEM),
            pl.BlockSpec(memory_space=pltpu.VMEM),
        ],
        out_specs=pl.BlockSpec(memory_space=pltpu.VMEM),
        scratch_shapes=[
            pltpu.VMEM((3, BH, D), jnp.float32),
            pltpu.VMEM((3, BH, D), jnp.float32),
            pltpu.SemaphoreType.DMA,
            pltpu.SemaphoreType.DMA,
        ],
        compiler_params=pltpu.CompilerParams(collective_id=0),
    )(K3, V3, Qf, bt, lens2)
    return res.reshape(B, 1, H, D)


def _dot(a, b):
    return jax.lax.dot_general(
        a, b, (((1,), (0,)), ((), ())), preferred_element_type=jnp.float32
    )


# device time: 12602 ns/iter; 1.2160x vs baseline; 1.2160x over previous
import jax
import jax.numpy as jnp
from jax import lax
from jax.experimental import pallas as pl
from jax.experimental.pallas import tpu as pltpu

P = 64
BS = 16
NK = P * BS
B = 8
H = 8
D = 64
HD = H * D
BH = B * H
NB = 64
SCALE = D ** -0.5
NEG = -1e30


def _dot(a, b):
    return jax.lax.dot_general(
        a, b, (((1,), (0,)), ((), ())), preferred_element_type=jnp.float32
    )


def kernel(Q, K, V, bt, lens):
    K2 = K.reshape(NK * H, D).astype(jnp.bfloat16).reshape(NK, HD)
    V2 = V.reshape(NK * H, D).astype(jnp.bfloat16).reshape(NK, HD)
    Qf = Q.reshape(BH, D)
    lens2 = lens.reshape(B, 1)

    def body(k_ref, v_ref, q_ref, bt_ref, lens_ref, out_ref,
             psend, precv, send_sem, recv_sem):
        mx = lax.axis_index("x")
        my = lax.axis_index("y")
        mz = lax.axis_index("z")
        x_nbr = (1 - mx, my, mz)

        barrier_sem = pltpu.get_barrier_semaphore()
        pl.semaphore_signal(barrier_sem, inc=1, device_id=x_nbr,
                            device_id_type=pl.DeviceIdType.MESH)

        e8 = (
            lax.broadcasted_iota(jnp.int32, (B, BH), 1) // H
            == lax.broadcasted_iota(jnp.int32, (B, BH), 0)
        ).astype(jnp.float32)
        g = (
            lax.broadcasted_iota(jnp.int32, (BH, HD), 0) % H
            == lax.broadcasted_iota(jnp.int32, (BH, HD), 1) // D
        ).astype(jnp.float32)
        rep = (
            lax.broadcasted_iota(jnp.int32, (NK, P), 0) // BS
            == lax.broadcasted_iota(jnp.int32, (NK, P), 1)
        ).astype(jnp.float32)
        i64 = (
            lax.broadcasted_iota(jnp.int32, (BH, BH), 0)
            == lax.broadcasted_iota(jnp.int32, (BH, BH), 1)
        ).astype(jnp.float32)

        qt = jax.lax.dot_general(
            q_ref[...], i64, (((0,), (0,)), ((), ())),
            preferred_element_type=jnp.float32,
        )
        maskq = (
            lax.broadcasted_iota(jnp.int32, (H, D, BH), 0)
            == lax.broadcasted_iota(jnp.int32, (H, D, BH), 2) % H
        ).astype(jnp.float32)
        qm = (jnp.broadcast_to(qt[None], (H, D, BH)) * maskq).reshape(
            HD, BH
        ).astype(jnp.bfloat16)

        pg = lax.broadcasted_iota(jnp.int32, (P, B, NB), 0) + mx * P
        slot = lax.broadcasted_iota(jnp.int32, (P, B, NB), 2)
        btv = bt_ref[...][None]
        lv = lens_ref[...][None]
        match = (btv == pg) & (slot < lv)
        c_pages = jnp.sum(match.astype(jnp.float32), axis=2)
        c_ab = _dot(rep, c_pages)
        c_cols = _dot(c_ab, e8)

        s_all = _dot(k_ref[...], qm) * SCALE
        s_m = jnp.where(c_cols > 0.0, s_all, NEG)

        m_cols = jnp.max(s_m, axis=0, keepdims=True)
        e = jnp.exp(s_m - m_cols)
        w = c_cols * e
        l_cols = jnp.sum(w, axis=0, keepdims=True)

        r = jax.lax.dot_general(
            w.astype(jnp.bfloat16), v_ref[...], (((0,), (0,)), ((), ())),
            preferred_element_type=jnp.float32,
        )
        acc_flat = _dot(e8, r * g)
        m_flat = _dot(jnp.broadcast_to(m_cols, (B, BH)) * e8, g)
        l_flat = _dot(jnp.broadcast_to(l_cols, (B, BH)) * e8, g)

        psend[0] = acc_flat
        psend[1] = m_flat
        psend[2] = l_flat

        pl.semaphore_wait(barrier_sem, 1)
        rdma = pltpu.make_async_remote_copy(
            src_ref=psend, dst_ref=precv,
            send_sem=send_sem, recv_sem=recv_sem,
            device_id=x_nbr, device_id_type=pl.DeviceIdType.MESH,
        )
        rdma.start()
        rdma.wait()

        acc0, m0, l0 = psend[0], psend[1], psend[2]
        acc1, m1, l1 = precv[0], precv[1], precv[2]
        m_new = jnp.maximum(m0, m1)
        e0 = jnp.exp(m0 - m_new)
        e1 = jnp.exp(m1 - m_new)
        out_ref[...] = (acc0 * e0 + acc1 * e1) / (l0 * e0 + l1 * e1)

    out_shape = jax.ShapeDtypeStruct((B, HD), jnp.float32)
    res = pl.pallas_call(
        body,
        out_shape=out_shape,
        in_specs=[
            pl.BlockSpec(memory_space=pltpu.VMEM),
            pl.BlockSpec(memory_space=pltpu.VMEM),
            pl.BlockSpec(memory_space=pltpu.VMEM),
            pl.BlockSpec(memory_space=pltpu.VMEM),
            pl.BlockSpec(memory_space=pltpu.VMEM),
        ],
        out_specs=pl.BlockSpec(memory_space=pltpu.VMEM),
        scratch_shapes=[
            pltpu.VMEM((3, B, HD), jnp.float32),
            pltpu.VMEM((3, B, HD), jnp.float32),
            pltpu.SemaphoreType.DMA,
            pltpu.SemaphoreType.DMA,
        ],
        compiler_params=pltpu.CompilerParams(collective_id=0),
    )(K2, V2, Qf, bt, lens2)
    return res.reshape(B, 1, H, D)
